# baseline (speedup 1.0000x reference)
import functools

import jax
import jax.numpy as jnp
from jax import lax
from jax.experimental import pallas as pl
from jax.experimental.pallas import tpu as pltpu
from jax.experimental.pallas import tpu_sc as plsc

B = 16384
D = 64
NT = 6
OFFSETS = (0, 2, 808, 1743, 2589, 3000)

_info = plsc.get_sparse_core_info()
_NC = _info.num_cores
_NS = _info.num_subcores
NW = _NC * _NS
NOUT = B * NT // NW
NIDX = 128
NSTR = NOUT // NIDX
NBUF = 12
AHEAD = 8

_mesh = plsc.VectorSubcoreMesh(core_axis_name="c", subcore_axis_name="s")


@functools.partial(
    pl.kernel,
    mesh=_mesh,
    compiler_params=pltpu.CompilerParams(use_tc_tiling_on_sc=False),
    out_type=jax.ShapeDtypeStruct((B * NT, D), jnp.float32),
    scratch_types=[
        pltpu.VMEM((NSTR, NIDX), jnp.int32),
        pltpu.VMEM((NBUF * NIDX, D), jnp.float32),
        pltpu.SemaphoreType.DMA,
        pltpu.SemaphoreType.DMA,
    ],
)
def _emb_kernel(cidx_h, wcat_h, out_h, cidx_v, rows_v, gsem, wsem):
    wid = lax.axis_index("s") * _NC + lax.axis_index("c")
    pltpu.sync_copy(cidx_h.at[pl.ds(wid * NSTR, NSTR)], cidx_v)
    base = wid * NOUT

    def buf(i):
        return rows_v.at[pl.ds((i % NBUF) * NIDX, NIDX)]

    def gfire(i):
        return pltpu.async_copy(wcat_h.at[cidx_v.at[i]], buf(i), gsem)

    def wfire(i):
        return pltpu.async_copy(buf(i), out_h.at[pl.ds(base + i * NIDX, NIDX)],
                                wsem)

    g = [None] * NSTR
    w = [None] * NSTR
    for i in range(AHEAD):
        g[i] = gfire(i)
    for i in range(NSTR):
        g[i].wait()
        w[i] = wfire(i)
        nxt = i + AHEAD
        if nxt < NSTR:
            if nxt - NBUF >= 0:
                w[nxt - NBUF].wait()
            g[nxt] = gfire(nxt)
    for i in range(max(0, NSTR - NBUF), NSTR):
        w[i].wait()


def kernel(cate, customer, brand, campaign, price, pids, id, W_cate,
           W_customer, W_brand, W_campaign, W_price, W_pids, W_id):
    wcat = jnp.concatenate(
        [W_pids, W_cate, W_customer, W_brand, W_campaign, W_price], axis=0)
    offs = jnp.asarray(OFFSETS, dtype=jnp.int32)
    cidx = (jnp.stack([pids, cate, customer, brand, campaign, price], axis=1)
            + offs[None, :]).reshape(B * NT // NIDX, NIDX)
    out = _emb_kernel(cidx, wcat)
    return out.reshape(B, NT, D)


# speedup vs baseline: 3.1700x; 3.1700x over previous
import functools

import jax
import jax.numpy as jnp
from jax import lax
from jax.experimental import pallas as pl
from jax.experimental.pallas import tpu as pltpu
from jax.experimental.pallas import tpu_sc as plsc

B = 16384
D = 64
NT = 6

V_PIDS, V_CATE, V_CUST, V_BRAND, V_CAMP, V_PRICE = 2, 806, 935, 846, 411, 11
A_STACK_ROWS = (V_PIDS, V_CATE, V_CAMP, V_PRICE)
A_OFF_W = (0, V_PIDS * D, (V_PIDS + V_CATE) * D, (V_PIDS + V_CATE + V_CAMP) * D)
TBL_WORDS = (V_PIDS + V_CATE + V_CAMP + V_PRICE) * D

N_CB = 2048
N_A = 1024
CB_CHUNK = 128
A_CHUNK = 32

_info = plsc.get_sparse_core_info()
_NC = _info.num_cores
_NS = _info.num_subcores

_mesh = plsc.VectorSubcoreMesh(core_axis_name="c", subcore_axis_name="s")


@functools.partial(
    pl.kernel,
    mesh=_mesh,
    compiler_params=pltpu.CompilerParams(use_tc_tiling_on_sc=False),
    out_type=jax.ShapeDtypeStruct((B, NT * D), jnp.float32),
    scratch_types=[
        pltpu.VMEM((TBL_WORDS,), jnp.float32),
        pltpu.VMEM((4, 1024), jnp.int32),
        pltpu.VMEM((2, CB_CHUNK, D), jnp.float32),
        pltpu.VMEM((2, A_CHUNK, 2 * D), jnp.float32),
        pltpu.VMEM((2, A_CHUNK, 2 * D), jnp.float32),
        pltpu.SemaphoreType.DMA,
    ],
)
def _emb_kernel(pids_h, cate_h, cust_h, brand_h, camp_h, price_h,
                wpids_h, wcate_h, wcust_h, wbrand_h, wcamp_h, wprice_h,
                out_h, tbl, idx4, obuf_cb, obuf01, obuf45, wsem):
    wid = lax.axis_index("s") * _NC + lax.axis_index("c")

    def gather_row(buf, p, i, idx, coloff):
        a = idx * D
        for k in range(D // 16):
            buf[p, i, pl.ds(coloff + k * 16, 16)] = tbl[pl.ds(a + k * 16, 16)]

    def gather16(buf, p, irow0, vidx, coloff, woff):
        for j in range(16):
            gather_row(buf, p, irow0 + j, vidx[j] + woff, coloff)

    def cb_group(idx2_h, wtbl_h, words, local, tcol):
        pltpu.sync_copy(wtbl_h, tbl.at[pl.ds(0, words)])
        pltpu.sync_copy(idx2_h.at[pl.ds(local * 2, 2)], idx4.at[pl.ds(0, 2)])
        nch = N_CB // CB_CHUNK

        def gather_chunk(c, slot):
            def body(g, carry):
                r = c * CB_CHUNK + g * 16
                vidx = idx4[r // 1024, pl.ds(lax.rem(r, 1024), 16)]
                gather16(obuf_cb, slot, g * 16, vidx, 0, 0)
                return carry

            lax.fori_loop(0, CB_CHUNK // 16, body, 0)

        def fire_write(c, slot):
            b0 = local * N_CB + c * CB_CHUNK
            return pltpu.async_copy(
                obuf_cb.at[slot],
                out_h.at[pl.ds(b0, CB_CHUNK), pl.ds(tcol, D)], wsem)

        for b in range(2):
            gather_chunk(b, b)
            fire_write(b, b)

        def drain(slot):
            pltpu.make_async_copy(
                obuf_cb.at[slot],
                out_h.at[pl.ds(local * N_CB, CB_CHUNK), pl.ds(tcol, D)],
                wsem).wait()

        def outer(i, carry):
            for b in range(2):
                c = 2 * i + 2 + b
                drain(b)
                gather_chunk(c, b)
                fire_write(c, b)
            return carry

        lax.fori_loop(0, (nch - 2) // 2, outer, 0)
        for b in range(2):
            drain(b)

    def a_group(local):
        srcs = (wpids_h, wcate_h, wcamp_h, wprice_h)
        for s in range(4):
            pltpu.sync_copy(srcs[s],
                            tbl.at[pl.ds(A_OFF_W[s], A_STACK_ROWS[s] * D)])
        idxs = (pids_h, cate_h, camp_h, price_h)
        for s in range(4):
            pltpu.sync_copy(idxs[s].at[pl.ds(local, 1)],
                            idx4.at[pl.ds(s, 1)])
        nch = N_A // A_CHUNK

        def gather_chunk(c, slot):
            def body(g, carry):
                r0 = c * A_CHUNK + g * 16
                gather16(obuf01, slot, g * 16, idx4[0, pl.ds(r0, 16)], 0,
                         A_OFF_W[0] // D)
                gather16(obuf01, slot, g * 16, idx4[1, pl.ds(r0, 16)], D,
                         A_OFF_W[1] // D)
                gather16(obuf45, slot, g * 16, idx4[2, pl.ds(r0, 16)], 0,
                         A_OFF_W[2] // D)
                gather16(obuf45, slot, g * 16, idx4[3, pl.ds(r0, 16)], D,
                         A_OFF_W[3] // D)
                return carry

            lax.fori_loop(0, A_CHUNK // 16, body, 0)

        def fire_writes(c, slot):
            b0 = local * N_A + c * A_CHUNK
            h0 = pltpu.async_copy(
                obuf01.at[slot],
                out_h.at[pl.ds(b0, A_CHUNK), pl.ds(0, 2 * D)], wsem)
            h1 = pltpu.async_copy(
                obuf45.at[slot],
                out_h.at[pl.ds(b0, A_CHUNK), pl.ds(4 * D, 2 * D)], wsem)
            return h0, h1

        def drain(slot):
            for buf in (obuf01, obuf45):
                pltpu.make_async_copy(
                    buf.at[slot],
                    out_h.at[pl.ds(local * N_A, A_CHUNK), pl.ds(0, 2 * D)],
                    wsem).wait()

        for b in range(2):
            gather_chunk(b, b)
            fire_writes(b, b)

        def outer(i, carry):
            for b in range(2):
                c = 2 * i + 2 + b
                drain(b)
                gather_chunk(c, b)
                fire_writes(c, b)
            return carry

        lax.fori_loop(0, (nch - 2) // 2, outer, 0)
        for b in range(2):
            drain(b)

    @pl.when(wid < 8)
    def _():
        cb_group(cust_h, wcust_h, V_CUST * D, wid, 2 * D)

    @pl.when((wid >= 8) & (wid < 16))
    def _():
        cb_group(brand_h, wbrand_h, V_BRAND * D, wid - 8, 3 * D)

    @pl.when(wid >= 16)
    def _():
        a_group(wid - 16)


def kernel(cate, customer, brand, campaign, price, pids, id, W_cate,
           W_customer, W_brand, W_campaign, W_price, W_pids, W_id):
    shp = (B // 1024, 1024)
    out = _emb_kernel(
        pids.reshape(shp), cate.reshape(shp), customer.reshape(shp),
        brand.reshape(shp), campaign.reshape(shp), price.reshape(shp),
        W_pids.reshape(-1), W_cate.reshape(-1), W_customer.reshape(-1),
        W_brand.reshape(-1), W_campaign.reshape(-1), W_price.reshape(-1))
    return out.reshape(B, NT, D)
